# baseline (speedup 1.0000x reference)
import functools

import jax
import jax.numpy as jnp
from jax import lax
from jax.experimental import pallas as pl
from jax.experimental.pallas import tpu as pltpu
from jax.experimental.pallas import tpu_sc as plsc

B = 16384
C = 128
NC = 2
NS = 16
NW = NC * NS
ROWS_W = B // NW
CHUNK = 128
NCHUNK = ROWS_W // CHUNK
LANES = 16

_mesh = plsc.VectorSubcoreMesh(core_axis_name="c", subcore_axis_name="s")


@functools.partial(
    pl.kernel,
    out_type=jax.ShapeDtypeStruct((B, C), jnp.float32),
    mesh=_mesh,
    scratch_types=[
        pltpu.VMEM((NCHUNK, CHUNK), jnp.int32),
        pltpu.VMEM((NCHUNK, CHUNK), jnp.int32),
        pltpu.VMEM((NCHUNK, CHUNK), jnp.int32),
        [pltpu.VMEM((CHUNK, C), jnp.float32) for _ in range(2)],
        [pltpu.VMEM((CHUNK, C), jnp.float32) for _ in range(2)],
        [pltpu.VMEM((CHUNK, C), jnp.float32) for _ in range(2)],
        [pltpu.SemaphoreType.DMA for _ in range(2)],
        pltpu.SemaphoreType.DMA,
    ],
)
def _embed_silu(cidx_hbm, sidx_hbm, tidx_hbm, ctab_hbm, stab_hbm, ttab_hbm,
                out_hbm, cidx_v, sidx_v, tidx_v, abuf, bbuf, cbuf, gsem, osem):
    wid = lax.axis_index("s") * NC + lax.axis_index("c")
    pltpu.sync_copy(cidx_hbm.at[wid], cidx_v)
    pltpu.sync_copy(sidx_hbm.at[wid], sidx_v)
    pltpu.sync_copy(tidx_hbm.at[wid], tidx_v)

    def start_gather(j):
        s = j % 2
        return (
            pltpu.async_copy(ctab_hbm.at[cidx_v.at[j]], abuf[s], gsem[s]),
            pltpu.async_copy(stab_hbm.at[sidx_v.at[j]], bbuf[s], gsem[s]),
            pltpu.async_copy(ttab_hbm.at[tidx_v.at[j]], cbuf[s], gsem[s]),
        )

    gathers = {0: start_gather(0)}
    outs = {}
    for j in range(NCHUNK):
        s = j % 2
        if j + 1 < NCHUNK:
            if j - 1 in outs:
                outs.pop(j - 1).wait()
            gathers[j + 1] = start_gather(j + 1)
        for d in gathers.pop(j):
            d.wait()

        @plsc.parallel_loop(0, CHUNK, step=1, unroll=2)
        def row_body(i):
            xs = []
            for c8 in range(C // LANES):
                sl = pl.ds(c8 * LANES, LANES)
                xs.append(abuf[s][i, sl] + bbuf[s][i, sl] + cbuf[s][i, sl])
            ts = [1.0 + jnp.exp(-x) for x in xs]
            for c8 in range(C // LANES):
                sl = pl.ds(c8 * LANES, LANES)
                abuf[s][i, sl] = xs[c8] / ts[c8]
        outs[j] = pltpu.async_copy(
            abuf[s], out_hbm.at[pl.ds(wid * ROWS_W + j * CHUNK, CHUNK)], osem)
    for j in sorted(outs):
        outs.pop(j).wait()


def kernel(charge, spin, task, charge_table, spin_table, task_table, bias):
    cidx = (charge + 100).reshape(NW, NCHUNK, CHUNK)
    sidx = spin.reshape(NW, NCHUNK, CHUNK)
    tidx = task.reshape(NW, NCHUNK, CHUNK)
    ttab = task_table + bias[None, :]
    return _embed_silu(cidx, sidx, tidx, charge_table, spin_table, ttab)


# speedup vs baseline: 2.0534x; 2.0534x over previous
import functools

import jax
import jax.numpy as jnp
from jax import lax
from jax.experimental import pallas as pl
from jax.experimental.pallas import tpu as pltpu
from jax.experimental.pallas import tpu_sc as plsc

B = 16384
C = 128
NC = 2
NS = 16
NW = NC * NS
ROWS_W = B // NW
CROWS = 256
NCH = ROWS_W // CROWS
LANES = 16

CV = 201
SV = 101
TV = 16

_mesh = plsc.VectorSubcoreMesh(core_axis_name="c", subcore_axis_name="s")


@functools.partial(
    pl.kernel,
    out_type=jax.ShapeDtypeStruct((B, C), jnp.float32),
    mesh=_mesh,
    compiler_params=pltpu.CompilerParams(needs_layout_passes=False),
    scratch_types=[
        pltpu.VMEM((CV * C,), jnp.float32),
        pltpu.VMEM((SV * C,), jnp.float32),
        pltpu.VMEM((TV * C,), jnp.float32),
        pltpu.VMEM((ROWS_W,), jnp.int32),
        pltpu.VMEM((ROWS_W,), jnp.int32),
        pltpu.VMEM((ROWS_W,), jnp.int32),
        [pltpu.VMEM((CROWS, C), jnp.float32) for _ in range(2)],
        pltpu.SemaphoreType.DMA,
        pltpu.SemaphoreType.DMA,
    ],
)
def _embed_silu(ctab_hbm, stab_hbm, ttab_hbm, cidx_hbm, sidx_hbm, tidx_hbm,
                out_hbm, ctab_v, stab_v, ttab_v, cidx_v, sidx_v, tidx_v,
                obuf, ssem, osem):
    wid = lax.axis_index("s") * NC + lax.axis_index("c")
    stage = (
        pltpu.async_copy(ctab_hbm, ctab_v, ssem),
        pltpu.async_copy(stab_hbm, stab_v, ssem),
        pltpu.async_copy(ttab_hbm, ttab_v, ssem),
        pltpu.async_copy(cidx_hbm.at[wid], cidx_v, ssem),
        pltpu.async_copy(sidx_hbm.at[wid], sidx_v, ssem),
        pltpu.async_copy(tidx_hbm.at[wid], tidx_v, ssem),
    )
    for d in stage:
        d.wait()

    iota = lax.broadcasted_iota(jnp.int32, (LANES,), 0)
    cols = [iota + (c0 * LANES) for c0 in range(C // LANES)]

    outs = {}
    for j in range(NCH):
        s = j % 2

        @plsc.parallel_loop(0, CROWS, step=1, unroll=2)
        def row_body(i):
            g = jnp.full((LANES,), j * CROWS, jnp.int32) + i
            ci = plsc.load_gather(cidx_v, [g]) * C
            si = plsc.load_gather(sidx_v, [g]) * C
            ti = plsc.load_gather(tidx_v, [g]) * C
            for c0 in range(C // LANES):
                a = plsc.load_gather(ctab_v, [ci + cols[c0]])
                b = plsc.load_gather(stab_v, [si + cols[c0]])
                t = plsc.load_gather(ttab_v, [ti + cols[c0]])
                x = a + b + t
                obuf[s][i, pl.ds(c0 * LANES, LANES)] = x / (1.0 + jnp.exp(-x))

        outs[j] = pltpu.async_copy(
            obuf[s], out_hbm.at[pl.ds(wid * ROWS_W + j * CROWS, CROWS)], osem)
    for j in sorted(outs):
        outs.pop(j).wait()


def kernel(charge, spin, task, charge_table, spin_table, task_table, bias):
    cidx = (charge + 100).reshape(NW, ROWS_W)
    sidx = spin.reshape(NW, ROWS_W)
    tidx = task.reshape(NW, ROWS_W)
    ttab = (task_table + bias[None, :]).reshape(-1)
    return _embed_silu(charge_table.reshape(-1), spin_table.reshape(-1), ttab,
                       cidx, sidx, tidx)


# speedup vs baseline: 2.0864x; 1.0161x over previous
import functools

import jax
import jax.numpy as jnp
from jax import lax
from jax.experimental import pallas as pl
from jax.experimental.pallas import tpu as pltpu
from jax.experimental.pallas import tpu_sc as plsc

B = 16384
C = 128
NC = 2
NS = 16
NW = NC * NS
ROWS_W = B // NW
CROWS = 256
NCH = ROWS_W // CROWS
LANES = 16

CV = 201
SV = 101
TV = 16

_mesh = plsc.VectorSubcoreMesh(core_axis_name="c", subcore_axis_name="s")


@functools.partial(
    pl.kernel,
    out_type=jax.ShapeDtypeStruct((B, C), jnp.float32),
    mesh=_mesh,
    compiler_params=pltpu.CompilerParams(needs_layout_passes=False),
    scratch_types=[
        pltpu.VMEM((CV * C,), jnp.float32),
        pltpu.VMEM((SV * C,), jnp.float32),
        pltpu.VMEM((TV * C,), jnp.float32),
        pltpu.VMEM((ROWS_W,), jnp.int32),
        pltpu.VMEM((ROWS_W,), jnp.int32),
        pltpu.VMEM((ROWS_W,), jnp.int32),
        [pltpu.VMEM((CROWS, C), jnp.float32) for _ in range(2)],
        pltpu.VMEM((C,), jnp.float32),
        pltpu.SemaphoreType.DMA,
        pltpu.SemaphoreType.DMA,
    ],
)
def _embed_silu(ctab_hbm, stab_hbm, ttab_hbm, cidx_hbm, sidx_hbm, tidx_hbm,
                bias_hbm, out_hbm, ctab_v, stab_v, ttab_v, cidx_v, sidx_v,
                tidx_v, obuf, bias_v, ssem, osem):
    wid = lax.axis_index("s") * NC + lax.axis_index("c")
    base = wid * ROWS_W
    stage = (
        pltpu.async_copy(ctab_hbm, ctab_v, ssem),
        pltpu.async_copy(stab_hbm, stab_v, ssem),
        pltpu.async_copy(ttab_hbm, ttab_v, ssem),
        pltpu.async_copy(bias_hbm, bias_v, ssem),
        pltpu.async_copy(cidx_hbm.at[pl.ds(base, ROWS_W)], cidx_v, ssem),
        pltpu.async_copy(sidx_hbm.at[pl.ds(base, ROWS_W)], sidx_v, ssem),
        pltpu.async_copy(tidx_hbm.at[pl.ds(base, ROWS_W)], tidx_v, ssem),
    )
    for d in stage:
        d.wait()

    for r in range(TV):
        for c0 in range(C // LANES):
            sl = pl.ds(r * C + c0 * LANES, LANES)
            bl = pl.ds(c0 * LANES, LANES)
            ttab_v[sl] = ttab_v[sl] + bias_v[bl]

    iota = lax.broadcasted_iota(jnp.int32, (LANES,), 0)
    cols = [iota + (c0 * LANES) for c0 in range(C // LANES)]
    ccols = [iota + (c0 * LANES + 100 * C) for c0 in range(C // LANES)]

    outs = {}
    for j in range(NCH):
        s = j % 2

        @plsc.parallel_loop(0, CROWS, step=1, unroll=2)
        def row_body(i):
            g = jnp.full((LANES,), j * CROWS, jnp.int32) + i
            ci = plsc.load_gather(cidx_v, [g]) * C
            si = plsc.load_gather(sidx_v, [g]) * C
            ti = plsc.load_gather(tidx_v, [g]) * C
            for c0 in range(C // LANES):
                a = plsc.load_gather(ctab_v, [ci + ccols[c0]])
                b = plsc.load_gather(stab_v, [si + cols[c0]])
                t = plsc.load_gather(ttab_v, [ti + cols[c0]])
                x = a + b + t
                obuf[s][i, pl.ds(c0 * LANES, LANES)] = x / (1.0 + jnp.exp(-x))

        outs[j] = pltpu.async_copy(
            obuf[s], out_hbm.at[pl.ds(base + j * CROWS, CROWS)], osem)
    for j in sorted(outs):
        outs.pop(j).wait()


def kernel(charge, spin, task, charge_table, spin_table, task_table, bias):
    return _embed_silu(charge_table.reshape(-1), spin_table.reshape(-1),
                       task_table.reshape(-1), charge, spin, task, bias)
